# baseline (speedup 1.0000x reference)
import functools

import jax
import jax.numpy as jnp
from jax import lax
from jax.experimental import pallas as pl
from jax.experimental.pallas import tpu as pltpu
from jax.experimental.pallas import tpu_sc as plsc

N_EXPERTS = 160
TOP_K = 6


_BT = 512


def _matmul_body(w_ref, x_ref, out_ref):
    out_ref[...] = lax.dot_general(
        w_ref[...], x_ref[...],
        dimension_numbers=(((1,), (1,)), ((), ())),
        preferred_element_type=jnp.float32,
    )


def _logits_t(x, weight):
    n_tok, h = x.shape
    grid = n_tok // _BT
    return pl.pallas_call(
        _matmul_body,
        grid=(grid,),
        in_specs=[
            pl.BlockSpec((N_EXPERTS, h), lambda i: (0, 0)),
            pl.BlockSpec((_BT, h), lambda i: (i, 0)),
        ],
        out_specs=pl.BlockSpec((N_EXPERTS, _BT), lambda i: (0, i)),
        out_shape=jax.ShapeDtypeStruct((N_EXPERTS, n_tok), jnp.float32),
    )(weight, x)


def _i32(c):
    return jnp.asarray(c, jnp.int32)


_SCALE = float(2 ** 20)


def _pack_key(v, e):
    iv = (v * _SCALE).astype(jnp.int32)
    return lax.bitwise_or(lax.shift_left(iv, _i32(8)),
                          jnp.broadcast_to(255 - e, (16,)).astype(jnp.int32))


def _insert(t, key):
    out = []
    cur = key
    for j in range(TOP_K):
        out.append(jnp.maximum(t[j], cur))
        if j < TOP_K - 1:
            cur = jnp.minimum(t[j], cur)
    return tuple(out)


_UNROLL = 8
_SLAB = 256


def _sc_topk(logits_t):
    n_exp, n_tok = logits_t.shape
    info = plsc.get_sparse_core_info()
    nc, ns = info.num_cores, info.num_subcores
    nw = nc * ns
    rows_per_w = n_tok // nw
    slab = min(_SLAB, rows_per_w)
    n_slab = rows_per_w // slab
    n_groups = slab // 16

    mesh = plsc.VectorSubcoreMesh(core_axis_name="c", subcore_axis_name="s")

    @functools.partial(
        pl.kernel,
        mesh=mesh,
        out_type=[
            jax.ShapeDtypeStruct((TOP_K, n_tok), jnp.int32),
            jax.ShapeDtypeStruct((TOP_K, n_tok), jnp.float32),
        ],
        scratch_types=[
            pltpu.VMEM((2, n_exp, slab), jnp.float32),
            pltpu.VMEM((TOP_K, rows_per_w), jnp.int32),
            pltpu.VMEM((TOP_K, rows_per_w), jnp.float32),
            pltpu.SemaphoreType.DMA,
            pltpu.SemaphoreType.DMA,
        ],
    )
    def k(logits_hbm, oidx_hbm, ow_hbm, buf, oi, ow, sem0, sem1):
        wid = lax.axis_index("s") * nc + lax.axis_index("c")
        sems = (sem0, sem1)

        def start(h, b):
            col0 = pl.multiple_of(wid * rows_per_w + h * slab, slab)
            return pltpu.async_copy(
                logits_hbm.at[:, pl.ds(col0, slab)], buf.at[b], sems[b])

        copies = [start(0, 0), start(1, 1) if n_slab > 1 else None]

        n_streams = 2

        for h in range(n_slab):
            b = h % 2
            copies[b].wait()

            def group_body(g, _):
                bases = [g * 16 * n_streams + 16 * q for q in range(n_streams)]
                ts = [tuple(jnp.full((16,), -(2 ** 31), jnp.int32)
                            for _ in range(TOP_K)) for _ in range(n_streams)]

                def exp_body(i, ts):
                    ts = list(ts)
                    for u in range(_UNROLL):
                        e = i * _UNROLL + u
                        for q in range(n_streams):
                            ts[q] = _insert(
                                ts[q], _pack_key(buf[b, e, pl.ds(bases[q], 16)], e))
                    return tuple(ts)

                ts = lax.fori_loop(0, n_exp // _UNROLL, exp_body, tuple(ts))

                for q in range(n_streams):
                    t = ts[q]
                    eidx = [255 - lax.bitwise_and(tj, _i32(255)) for tj in t]
                    vals = [lax.shift_right_arithmetic(tj, _i32(8)).astype(jnp.float32)
                            * (1.0 / _SCALE) for tj in t]
                    exps = [jnp.ones((16,), jnp.float32)]
                    exps += [jnp.exp(vals[j] - vals[0]) for j in range(1, TOP_K)]
                    s = exps[0]
                    for j in range(1, TOP_K):
                        s = s + exps[j]
                    r = 1.0 / s
                    row_local = h * slab + bases[q]
                    for j in range(TOP_K):
                        oi[j, pl.ds(row_local, 16)] = eidx[j]
                        ow[j, pl.ds(row_local, 16)] = exps[j] * r
                return 0

            lax.fori_loop(0, n_groups // n_streams, group_body, 0)
            if h + 2 < n_slab:
                copies[b] = start(h + 2, b)

        out0 = pl.multiple_of(wid * rows_per_w, rows_per_w)
        pltpu.sync_copy(oi, oidx_hbm.at[:, pl.ds(out0, rows_per_w)])
        pltpu.sync_copy(ow, ow_hbm.at[:, pl.ds(out0, rows_per_w)])

    return k(logits_t)


def kernel(hidden_states, weight):
    b, s, h = hidden_states.shape
    n_tok = b * s
    x = hidden_states.reshape(n_tok, h)
    logits_t = _logits_t(x, weight)
    idx_t, w_t = _sc_topk(logits_t)
    return idx_t.T, w_t.T


# speedup vs baseline: 1.0099x; 1.0099x over previous
import functools

import jax
import jax.numpy as jnp
from jax import lax
from jax.experimental import pallas as pl
from jax.experimental.pallas import tpu as pltpu
from jax.experimental.pallas import tpu_sc as plsc

N_EXPERTS = 160
TOP_K = 6


_BT = 512


def _matmul_body(w_ref, x_ref, out_ref):
    out_ref[...] = lax.dot_general(
        w_ref[...], x_ref[...],
        dimension_numbers=(((1,), (1,)), ((), ())),
        preferred_element_type=jnp.float32,
    )


def _logits_t(x, weight):
    n_tok, h = x.shape
    grid = n_tok // _BT
    return pl.pallas_call(
        _matmul_body,
        grid=(grid,),
        in_specs=[
            pl.BlockSpec((N_EXPERTS, h), lambda i: (0, 0)),
            pl.BlockSpec((_BT, h), lambda i: (i, 0)),
        ],
        out_specs=pl.BlockSpec((N_EXPERTS, _BT), lambda i: (0, i)),
        out_shape=jax.ShapeDtypeStruct((N_EXPERTS, n_tok), jnp.float32),
    )(weight, x)


def _i32(c):
    return jnp.asarray(c, jnp.int32)


_SCALE = float(2 ** 20)


def _pack_key(v, e):
    iv = (v * _SCALE).astype(jnp.int32)
    return lax.bitwise_or(lax.shift_left(iv, _i32(8)),
                          jnp.broadcast_to(255 - e, (16,)).astype(jnp.int32))


def _insert(t, key):
    out = []
    cur = key
    for j in range(TOP_K):
        out.append(jnp.maximum(t[j], cur))
        if j < TOP_K - 1:
            cur = jnp.minimum(t[j], cur)
    return tuple(out)


_UNROLL = 8
_SLAB = 256


def _sc_topk(logits_t):
    n_exp, n_tok = logits_t.shape
    info = plsc.get_sparse_core_info()
    nc, ns = info.num_cores, info.num_subcores
    nw = nc * ns
    rows_per_w = n_tok // nw
    slab = min(_SLAB, rows_per_w)
    n_slab = rows_per_w // slab
    n_groups = slab // 16

    mesh = plsc.VectorSubcoreMesh(core_axis_name="c", subcore_axis_name="s")

    @functools.partial(
        pl.kernel,
        mesh=mesh,
        out_type=[
            jax.ShapeDtypeStruct((TOP_K, n_tok), jnp.int32),
            jax.ShapeDtypeStruct((TOP_K, n_tok), jnp.float32),
        ],
        scratch_types=[
            pltpu.VMEM((2, n_exp, slab), jnp.float32),
            pltpu.VMEM((TOP_K, rows_per_w), jnp.int32),
            pltpu.VMEM((TOP_K, rows_per_w), jnp.float32),
            pltpu.SemaphoreType.DMA,
            pltpu.SemaphoreType.DMA,
        ],
    )
    def k(logits_hbm, oidx_hbm, ow_hbm, buf, oi, ow, sem0, sem1):
        wid = lax.axis_index("s") * nc + lax.axis_index("c")
        sems = (sem0, sem1)

        def start(h, b):
            col0 = pl.multiple_of(wid * rows_per_w + h * slab, slab)
            return pltpu.async_copy(
                logits_hbm.at[:, pl.ds(col0, slab)], buf.at[b], sems[b])

        copies = [start(0, 0), start(1, 1) if n_slab > 1 else None]

        n_streams = 1

        for h in range(n_slab):
            b = h % 2
            copies[b].wait()

            def group_body(g, _):
                bases = [g * 16 * n_streams + 16 * q for q in range(n_streams)]
                ts = [tuple(jnp.full((16,), -(2 ** 31), jnp.int32)
                            for _ in range(TOP_K)) for _ in range(n_streams)]

                def exp_body(i, ts):
                    ts = list(ts)
                    for u in range(_UNROLL):
                        e = i * _UNROLL + u
                        for q in range(n_streams):
                            ts[q] = _insert(
                                ts[q], _pack_key(buf[b, e, pl.ds(bases[q], 16)], e))
                    return tuple(ts)

                ts = lax.fori_loop(0, n_exp // _UNROLL, exp_body, tuple(ts))

                for q in range(n_streams):
                    t = ts[q]
                    eidx = [255 - lax.bitwise_and(tj, _i32(255)) for tj in t]
                    vals = [lax.shift_right_arithmetic(tj, _i32(8)).astype(jnp.float32)
                            * (1.0 / _SCALE) for tj in t]
                    exps = [jnp.ones((16,), jnp.float32)]
                    exps += [jnp.exp(vals[j] - vals[0]) for j in range(1, TOP_K)]
                    s = exps[0]
                    for j in range(1, TOP_K):
                        s = s + exps[j]
                    r = 1.0 / s
                    row_local = h * slab + bases[q]
                    for j in range(TOP_K):
                        oi[j, pl.ds(row_local, 16)] = eidx[j]
                        ow[j, pl.ds(row_local, 16)] = exps[j] * r
                return 0

            lax.fori_loop(0, n_groups // n_streams, group_body, 0)
            if h + 2 < n_slab:
                copies[b] = start(h + 2, b)

        out0 = pl.multiple_of(wid * rows_per_w, rows_per_w)
        pltpu.sync_copy(oi, oidx_hbm.at[:, pl.ds(out0, rows_per_w)])
        pltpu.sync_copy(ow, ow_hbm.at[:, pl.ds(out0, rows_per_w)])

    return k(logits_t)


def kernel(hidden_states, weight):
    b, s, h = hidden_states.shape
    n_tok = b * s
    x = hidden_states.reshape(n_tok, h)
    logits_t = _logits_t(x, weight)
    idx_t, w_t = _sc_topk(logits_t)
    return idx_t.T, w_t.T
